# Initial kernel scaffold; baseline (speedup 1.0000x reference)
import jax
import jax.numpy as jnp
from jax.experimental import pallas as pl


def kernel(x, token_embedding):
    raise NotImplementedError("write your pallas kernel here")


# speedup vs baseline: 5.7873x; 5.7873x over previous
import functools

import jax
import jax.numpy as jnp
from jax import lax
from jax.experimental import pallas as pl
from jax.experimental.pallas import tpu as pltpu
from jax.experimental.pallas import tpu_sc as plsc

V = 53
D = 64
BF = 16384 * 200
GRP = 128
NGRP = BF // GRP
NC = 2
NS = 16
NW = NC * NS
GPW = NGRP // NW
G = 4
NCH = GPW // G
NBUF = 2

_mesh = plsc.VectorSubcoreMesh(core_axis_name="c", subcore_axis_name="s")


@functools.partial(
    pl.kernel,
    mesh=_mesh,
    compiler_params=pltpu.CompilerParams(use_tc_tiling_on_sc=False),
    out_type=jax.ShapeDtypeStruct((NGRP, GRP, D), jnp.float32),
    scratch_types=[
        pltpu.VMEM_SHARED((V, D), jnp.float32),
        pltpu.VMEM((NBUF, G, GRP), jnp.int32),
        pltpu.VMEM((NBUF, G, GRP, D), jnp.float32),
        pltpu.SemaphoreType.DMA,
        pltpu.SemaphoreType.DMA,
        pltpu.SemaphoreType.DMA,
        pltpu.SemaphoreType.DMA,
        pltpu.SemaphoreType.DMA,
        pltpu.SemaphoreType.DMA,
    ],
)
def _sc_gather(x_hbm, tab_hbm, out_hbm, tab_sh, idx_v, rows_v,
               is0, is1, gs0, gs1, os0, os1):
    idx_sem = (is0, is1)
    g_sem = (gs0, gs1)
    o_sem = (os0, os1)
    cid = lax.axis_index("c")
    sid = lax.axis_index("s")
    wid = sid * NC + cid
    base = wid * GPW

    @pl.when(sid == 0)
    def _():
        pltpu.sync_copy(tab_hbm, tab_sh)
    plsc.subcore_barrier()

    for b in range(NBUF):
        pltpu.make_async_copy(
            x_hbm.at[pl.ds(base + b * G, G)], idx_v.at[b], idx_sem[b]
        ).start()

    def chunk_body(c, b):
        @pl.when(c >= NBUF)
        def _():
            pltpu.make_async_copy(
                rows_v.at[b], out_hbm.at[pl.ds(0, G)], o_sem[b]
            ).wait()
        pltpu.make_async_copy(
            x_hbm.at[pl.ds(0, G)], idx_v.at[b], idx_sem[b]
        ).wait()
        for j in range(G):
            pltpu.make_async_copy(
                tab_sh.at[idx_v.at[b, j]], rows_v.at[b, j], g_sem[b]
            ).start()
        for j in range(G):
            pltpu.make_async_copy(
                tab_sh.at[idx_v.at[b, j]], rows_v.at[b, j], g_sem[b]
            ).wait()
        @pl.when(c + NBUF < NCH)
        def _():
            pltpu.make_async_copy(
                x_hbm.at[pl.ds(base + (c + NBUF) * G, G)],
                idx_v.at[b],
                idx_sem[b],
            ).start()
        pltpu.make_async_copy(
            rows_v.at[b], out_hbm.at[pl.ds(base + c * G, G)], o_sem[b]
        ).start()

    def pair_body(g2, carry):
        for b in range(NBUF):
            chunk_body(g2 * NBUF + b, b)
        return carry

    lax.fori_loop(0, NCH // NBUF, pair_body, 0)

    for b in range(NBUF):
        pltpu.make_async_copy(
            rows_v.at[b], out_hbm.at[pl.ds(0, G)], o_sem[b]
        ).wait()


def kernel(x, token_embedding):
    xf = x.reshape(BF).astype(jnp.int32).reshape(NGRP, GRP)
    out = _sc_gather(xf, token_embedding.astype(jnp.float32))
    return out.reshape(x.shape[0], x.shape[1], D)
